# Initial kernel scaffold; baseline (speedup 1.0000x reference)
import jax
import jax.numpy as jnp
from jax.experimental import pallas as pl


def kernel(x, We, be, W1, b1, W2, b2):
    raise NotImplementedError("write your pallas kernel here")


# speedup vs baseline: 2.5557x; 2.5557x over previous
import jax
import jax.numpy as jnp
from jax.experimental import pallas as pl

_TOKENS, _DIN, _DOUT, _E = 4096, 768, 1024, 8
_H = 1536
_BLK = 256


def _dot(a, b):
    return jax.lax.dot_general(a, b, (((1,), (0,)), ((), ())),
                               preferred_element_type=jnp.float32)


def _fused_body(x_ref, w1_ref, b1_ref, w2_ref, b2_ref, we_ref, be_ref, o_ref):
    x = x_ref[...]
    h = jnp.maximum(_dot(x, w1_ref[...]) + b1_ref[...], 0.0)
    s = _dot(h, w2_ref[...]) + b2_ref[...]
    lane = jax.lax.broadcasted_iota(jnp.int32, s.shape, 1)
    s = jnp.where(lane < _E, s, -1e30)
    mx = jnp.max(s, axis=1, keepdims=True)
    done = jnp.zeros((x.shape[0], 1), dtype=jnp.bool_)
    acc = jnp.zeros((x.shape[0], _DOUT), dtype=jnp.float32)
    for e in range(_E):
        hit = jnp.logical_and(s[:, e:e + 1] == mx, jnp.logical_not(done))
        done = jnp.logical_or(done, hit)
        pe = _dot(x, we_ref[:, e * _DOUT:(e + 1) * _DOUT])
        pe = pe + be_ref[:, e * _DOUT:(e + 1) * _DOUT]
        acc = acc + jnp.where(hit, pe, 0.0)
    o_ref[...] = acc


@jax.jit
def kernel(x, We, be, W1, b1, W2, b2):
    n = x.shape[0]
    w2p = jnp.zeros((_H, 128), W2.dtype).at[:, :_E].set(W2)
    b2p = jnp.zeros((1, 128), b2.dtype).at[0, :_E].set(b2)
    b1r = b1.reshape(1, _H)
    ber = be.reshape(1, _E * _DOUT)
    grid = (n // _BLK,)
    return pl.pallas_call(
        _fused_body,
        grid=grid,
        in_specs=[
            pl.BlockSpec((_BLK, _DIN), lambda b: (b, 0)),
            pl.BlockSpec((_DIN, _H), lambda b: (0, 0)),
            pl.BlockSpec((1, _H), lambda b: (0, 0)),
            pl.BlockSpec((_H, 128), lambda b: (0, 0)),
            pl.BlockSpec((1, 128), lambda b: (0, 0)),
            pl.BlockSpec((_DIN, _E * _DOUT), lambda b: (0, 0)),
            pl.BlockSpec((1, _E * _DOUT), lambda b: (0, 0)),
        ],
        out_specs=pl.BlockSpec((_BLK, _DOUT), lambda b: (b, 0)),
        out_shape=jax.ShapeDtypeStruct((n, _DOUT), jnp.float32),
    )(x, W1, b1r, w2p, b2p, We, ber)
